# Initial kernel scaffold; baseline (speedup 1.0000x reference)
import jax
import jax.numpy as jnp
from jax.experimental import pallas as pl


def kernel(x, edge_attr, edge_index, u, We, be, Wv, bv, Wg, bg):
    raise NotImplementedError("write your pallas kernel here")


# speedup vs baseline: 5.7540x; 5.7540x over previous
import functools

import jax
import jax.numpy as jnp
from jax import lax
from jax.experimental import pallas as pl
from jax.experimental.pallas import tpu as pltpu
from jax.experimental.pallas import tpu_sc as plsc

NC = 2
NS = 16
NW = NC * NS
GSZ = 128


def _proj_body(x_ref, ws_ref, wd_ref, os_ref, od_ref):
    xb = x_ref[...]
    os_ref[...] = jnp.dot(xb, ws_ref[...], preferred_element_type=jnp.float32)
    od_ref[...] = jnp.dot(xb, wd_ref[...], preferred_element_type=jnp.float32)


def _node_proj(x, Wes, Wed):
    n, d = x.shape
    de = Wes.shape[1]
    nb = 5
    r = n // nb
    return pl.pallas_call(
        _proj_body,
        grid=(nb,),
        in_specs=[
            pl.BlockSpec((r, d), lambda i: (i, 0)),
            pl.BlockSpec((d, de), lambda i: (0, 0)),
            pl.BlockSpec((d, de), lambda i: (0, 0)),
        ],
        out_specs=[
            pl.BlockSpec((r, de), lambda i: (i, 0)),
            pl.BlockSpec((r, de), lambda i: (i, 0)),
        ],
        out_shape=[
            jax.ShapeDtypeStruct((n, de), jnp.float32),
            jax.ShapeDtypeStruct((n, de), jnp.float32),
        ],
    )(x, Wes, Wed)


def _ebase_body(ea_ref, wee_ref, u_ref, weu_ref, be_ref, o_ref):
    cu = jnp.dot(u_ref[...], weu_ref[...], preferred_element_type=jnp.float32)
    o_ref[...] = (
        jnp.dot(ea_ref[...], wee_ref[...], preferred_element_type=jnp.float32)
        + cu + be_ref[...]
    )


def _edge_base(edge_attr, Wee, u2, Weu, be2):
    e, de = edge_attr.shape
    du = u2.shape[1]
    nb = 20
    r = e // nb
    return pl.pallas_call(
        _ebase_body,
        grid=(nb,),
        in_specs=[
            pl.BlockSpec((r, de), lambda i: (i, 0)),
            pl.BlockSpec((de, de), lambda i: (0, 0)),
            pl.BlockSpec((1, du), lambda i: (0, 0)),
            pl.BlockSpec((du, de), lambda i: (0, 0)),
            pl.BlockSpec((1, de), lambda i: (0, 0)),
        ],
        out_specs=pl.BlockSpec((r, de), lambda i: (i, 0)),
        out_shape=jax.ShapeDtypeStruct((e, de), jnp.float32),
    )(edge_attr, Wee, u2, Weu, be2)


def _edge_sc(xs, xd, e_base, src2, dst2, n, e, de):
    g = e // GSZ
    gmax = ((g + NW - 1) // NW + 7) // 8 * 8
    n_pad = ((n + 8 * NS - 1) // (8 * NS)) * 8 * NS
    rpt = n_pad // NS

    mesh = plsc.VectorSubcoreMesh(
        core_axis_name="c", subcore_axis_name="s", num_cores=NC, num_subcores=NS)

    @functools.partial(
        pl.kernel,
        mesh=mesh,
        compiler_params=pltpu.CompilerParams(use_tc_tiling_on_sc=False),
        out_type=[
            jax.ShapeDtypeStruct((e, de), jnp.float32),
            jax.ShapeDtypeStruct((NC, n_pad, de), jnp.float32),
            jax.ShapeDtypeStruct((NC, n_pad, de), jnp.float32),
        ],
        scratch_types=[
            pltpu.VMEM((gmax, GSZ), jnp.int32),
            pltpu.VMEM((gmax, GSZ), jnp.int32),
            pltpu.VMEM((GSZ, de), jnp.float32),
            pltpu.VMEM((GSZ, de), jnp.float32),
            pltpu.VMEM((GSZ, de), jnp.float32),
            pltpu.VMEM((rpt, de), jnp.float32),
            pltpu.VMEM_SHARED((n_pad, de), jnp.float32),
            pltpu.VMEM_SHARED((n_pad, de), jnp.float32),
            pltpu.SemaphoreType.DMA,
        ],
    )
    def body(xs_h, xd_h, eb_h, src_h, dst_h, enew_h, ain_h, aout_h,
             si_v, di_v, gs_v, gd_v, eb_v, st_v, ain_s, aout_s, sem):
        c = lax.axis_index("c")
        s = lax.axis_index("s")
        w = c * NS + s
        g0 = gmax * w
        ng = jnp.clip(g - g0, 0, gmax)

        def zrow(i, _):
            st_v[i, :] = jnp.zeros((de,), jnp.float32)
            return 0
        lax.fori_loop(0, rpt, zrow, 0)
        pltpu.sync_copy(st_v, ain_s.at[pl.ds(s * rpt, rpt)])
        pltpu.sync_copy(st_v, aout_s.at[pl.ds(s * rpt, rpt)])
        plsc.subcore_barrier()

        pltpu.sync_copy(src_h.at[pl.ds(g0, gmax)], si_v)
        pltpu.sync_copy(dst_h.at[pl.ds(g0, gmax)], di_v)

        def grp(j, _):
            row0 = (g0 + j) * GSZ
            d1 = pltpu.async_copy(xs_h.at[si_v.at[j]], gs_v, sem)
            d2 = pltpu.async_copy(xd_h.at[di_v.at[j]], gd_v, sem)
            pltpu.sync_copy(eb_h.at[pl.ds(row0, GSZ)], eb_v)
            d1.wait()
            d2.wait()

            def rowf(r, _):
                for k in range(4):
                    i = r * 4 + k
                    v = eb_v[i, :] + gs_v[i, :] + gd_v[i, :]
                    eb_v[i, :] = jnp.maximum(v, 0.0)
                return 0
            lax.fori_loop(0, GSZ // 4, rowf, 0)

            pltpu.sync_copy(eb_v, enew_h.at[pl.ds(row0, GSZ)])
            pltpu.sync_copy(eb_v, ain_s.at[di_v.at[j]], add=True)
            pltpu.sync_copy(eb_v, aout_s.at[si_v.at[j]], add=True)
            return 0
        lax.fori_loop(0, ng, grp, 0)
        plsc.subcore_barrier()

        pltpu.sync_copy(ain_s.at[pl.ds(s * rpt, rpt)], st_v)
        pltpu.sync_copy(st_v, ain_h.at[c, pl.ds(s * rpt, rpt)])
        pltpu.sync_copy(aout_s.at[pl.ds(s * rpt, rpt)], st_v)
        pltpu.sync_copy(st_v, aout_h.at[c, pl.ds(s * rpt, rpt)])

    return body(xs, xd, e_base, src2, dst2)


def _node_body(n_nodes, n_edges,
               x_ref, ai_ref, ao_ref, wvx_ref, wvi_ref, wvo_ref, u_ref,
               wvu_ref, bv_ref, wgx_ref, wge_ref, wgu_ref, bg_ref,
               xn_ref, un_ref, accx, acce):
    i = pl.program_id(0)
    nb = pl.num_programs(0)

    @pl.when(i == 0)
    def _():
        accx[...] = jnp.zeros_like(accx)
        acce[...] = jnp.zeros_like(acce)

    a_in = ai_ref[0] + ai_ref[1]
    a_out = ao_ref[0] + ao_ref[1]
    cv = jnp.dot(u_ref[...], wvu_ref[...], preferred_element_type=jnp.float32)
    h = (jnp.dot(x_ref[...], wvx_ref[...], preferred_element_type=jnp.float32)
         + jnp.dot(a_in, wvi_ref[...], preferred_element_type=jnp.float32)
         + jnp.dot(a_out, wvo_ref[...], preferred_element_type=jnp.float32)
         + cv + bv_ref[...])
    xn = jnp.maximum(h, 0.0)
    xn_ref[...] = xn
    accx[...] += jnp.sum(xn, axis=0, keepdims=True)
    acce[...] += jnp.sum(a_in, axis=0, keepdims=True)

    @pl.when(i == nb - 1)
    def _():
        mx = accx[...] * (1.0 / n_nodes)
        me = acce[...] * (1.0 / n_edges)
        gg = (jnp.dot(mx, wgx_ref[...], preferred_element_type=jnp.float32)
              + jnp.dot(me, wge_ref[...], preferred_element_type=jnp.float32)
              + jnp.dot(u_ref[...], wgu_ref[...], preferred_element_type=jnp.float32)
              + bg_ref[...])
        un_ref[...] = jnp.maximum(gg, 0.0)


def _node_global(x, ain2, aout2, Wvx, Wvi, Wvo, u2, Wvu, bv2,
                 Wgx, Wge, Wgu, bg2, n_edges):
    n, d = x.shape
    de = Wvi.shape[0]
    du = u2.shape[1]
    nb = 5
    r = n // nb
    const = lambda i: (0, 0)
    return pl.pallas_call(
        functools.partial(_node_body, n, n_edges),
        grid=(nb,),
        in_specs=[
            pl.BlockSpec((r, d), lambda i: (i, 0)),
            pl.BlockSpec((NC, r, de), lambda i: (0, i, 0)),
            pl.BlockSpec((NC, r, de), lambda i: (0, i, 0)),
            pl.BlockSpec((d, d), const),
            pl.BlockSpec((de, d), const),
            pl.BlockSpec((de, d), const),
            pl.BlockSpec((1, du), const),
            pl.BlockSpec((du, d), const),
            pl.BlockSpec((1, d), const),
            pl.BlockSpec((d, du), const),
            pl.BlockSpec((de, du), const),
            pl.BlockSpec((du, du), const),
            pl.BlockSpec((1, du), const),
        ],
        out_specs=[
            pl.BlockSpec((r, d), lambda i: (i, 0)),
            pl.BlockSpec((1, du), const),
        ],
        out_shape=[
            jax.ShapeDtypeStruct((n, d), jnp.float32),
            jax.ShapeDtypeStruct((1, du), jnp.float32),
        ],
        scratch_shapes=[
            pltpu.VMEM((1, d), jnp.float32),
            pltpu.VMEM((1, de), jnp.float32),
        ],
    )(x, ain2, aout2, Wvx, Wvi, Wvo, u2, Wvu, bv2, Wgx, Wge, Wgu, bg2)


def kernel(x, edge_attr, edge_index, u, We, be, Wv, bv, Wg, bg):
    n, d = x.shape
    e, de = edge_attr.shape
    du = u.shape[0]

    Wee, Wes, Wed, Weu = We[:de], We[de:de + d], We[de + d:de + 2 * d], We[de + 2 * d:]
    Wvx, Wvi, Wvo, Wvu = Wv[:d], Wv[d:d + de], Wv[d + de:d + 2 * de], Wv[d + 2 * de:]
    Wgx, Wge, Wgu = Wg[:d], Wg[d:d + de], Wg[d + de:]
    u2 = u.reshape(1, du)
    be2 = be.reshape(1, de)
    bv2 = bv.reshape(1, d)
    bg2 = bg.reshape(1, du)

    g = e // GSZ
    gmax = ((g + NW - 1) // NW + 7) // 8 * 8
    pad = gmax * NW - g
    src2 = jnp.pad(edge_index[0].reshape(g, GSZ), ((0, pad), (0, 0)))
    dst2 = jnp.pad(edge_index[1].reshape(g, GSZ), ((0, pad), (0, 0)))

    xs, xd = _node_proj(x, Wes, Wed)
    e_base = _edge_base(edge_attr, Wee, u2, Weu, be2)
    e_new, ain2, aout2 = _edge_sc(xs, xd, e_base, src2, dst2, n, e, de)
    ain2 = ain2[:, :n]
    aout2 = aout2[:, :n]
    x_new, u_new = _node_global(x, ain2, aout2, Wvx, Wvi, Wvo, u2, Wvu, bv2,
                                Wgx, Wge, Wgu, bg2, e)
    return x_new, e_new, u_new.reshape(du)


# speedup vs baseline: 7.6473x; 1.3290x over previous
import functools

import jax
import jax.numpy as jnp
from jax import lax
from jax.experimental import pallas as pl
from jax.experimental.pallas import tpu as pltpu
from jax.experimental.pallas import tpu_sc as plsc

NC = 2
NS = 16
NW = NC * NS
GSZ = 128


def _proj_body(x_ref, ws_ref, wd_ref, os_ref, od_ref):
    xb = x_ref[...]
    os_ref[...] = jnp.dot(xb, ws_ref[...], preferred_element_type=jnp.float32)
    od_ref[...] = jnp.dot(xb, wd_ref[...], preferred_element_type=jnp.float32)


def _node_proj(x, Wes, Wed):
    n, d = x.shape
    de = Wes.shape[1]
    nb = 5
    r = n // nb
    return pl.pallas_call(
        _proj_body,
        grid=(nb,),
        in_specs=[
            pl.BlockSpec((r, d), lambda i: (i, 0)),
            pl.BlockSpec((d, de), lambda i: (0, 0)),
            pl.BlockSpec((d, de), lambda i: (0, 0)),
        ],
        out_specs=[
            pl.BlockSpec((r, de), lambda i: (i, 0)),
            pl.BlockSpec((r, de), lambda i: (i, 0)),
        ],
        out_shape=[
            jax.ShapeDtypeStruct((n, de), jnp.float32),
            jax.ShapeDtypeStruct((n, de), jnp.float32),
        ],
    )(x, Wes, Wed)


def _ebase_body(ea_ref, w8_ref, u_ref, weu8_ref, be8_ref, o_ref):
    cu8 = jnp.dot(u_ref[...], weu8_ref[...], preferred_element_type=jnp.float32)
    o_ref[...] = (
        jnp.dot(ea_ref[...], w8_ref[...], preferred_element_type=jnp.float32)
        + cu8 + be8_ref[...]
    )


def _edge_base(ea2, W8, u2, Weu8, be8):
    ep, _ = ea2.shape
    du = u2.shape[1]
    nb = 20
    rp = ep // nb
    return pl.pallas_call(
        _ebase_body,
        grid=(nb,),
        in_specs=[
            pl.BlockSpec((rp, 128), lambda i: (i, 0)),
            pl.BlockSpec((128, 128), lambda i: (0, 0)),
            pl.BlockSpec((1, du), lambda i: (0, 0)),
            pl.BlockSpec((du, 128), lambda i: (0, 0)),
            pl.BlockSpec((1, 128), lambda i: (0, 0)),
        ],
        out_specs=pl.BlockSpec((rp, 128), lambda i: (i, 0)),
        out_shape=jax.ShapeDtypeStruct((ep, 128), jnp.float32),
    )(ea2, W8, u2, Weu8, be8)


def _edge_sc(xs, xd, e_base, src2, dst2, n, e, de):
    g = e // GSZ
    gmax = ((g + NW - 1) // NW + 7) // 8 * 8
    n_pad = ((n + 8 * NS - 1) // (8 * NS)) * 8 * NS
    rpt = n_pad // NS

    mesh = plsc.VectorSubcoreMesh(
        core_axis_name="c", subcore_axis_name="s", num_cores=NC, num_subcores=NS)

    gp = GSZ * de // 128

    @functools.partial(
        pl.kernel,
        mesh=mesh,
        compiler_params=pltpu.CompilerParams(use_tc_tiling_on_sc=False),
        out_type=[
            jax.ShapeDtypeStruct((e * de // 128, 128), jnp.float32),
            jax.ShapeDtypeStruct((NC, n_pad, de), jnp.float32),
            jax.ShapeDtypeStruct((NC, n_pad, de), jnp.float32),
        ],
        scratch_types=[
            pltpu.VMEM((gmax, GSZ), jnp.int32),
            pltpu.VMEM((gmax, GSZ), jnp.int32),
            pltpu.VMEM((GSZ, de), jnp.float32),
            pltpu.VMEM((GSZ, de), jnp.float32),
            pltpu.VMEM((gp, 128), jnp.float32),
            pltpu.VMEM((GSZ, de), jnp.float32),
            pltpu.VMEM((rpt, de), jnp.float32),
            pltpu.VMEM_SHARED((n_pad, de), jnp.float32),
            pltpu.VMEM_SHARED((n_pad, de), jnp.float32),
            pltpu.SemaphoreType.DMA,
        ],
    )
    def body(xs_h, xd_h, eb_h, src_h, dst_h, enew_h, ain_h, aout_h,
             si_v, di_v, gs_v, gd_v, eb_v, en_v, st_v, ain_s, aout_s, sem):
        c = lax.axis_index("c")
        s = lax.axis_index("s")
        w = c * NS + s
        g0 = gmax * w
        ng = jnp.clip(g - g0, 0, gmax)

        def zrow(i, _):
            st_v[i, :] = jnp.zeros((de,), jnp.float32)
            return 0
        lax.fori_loop(0, rpt, zrow, 0)
        pltpu.sync_copy(st_v, ain_s.at[pl.ds(s * rpt, rpt)])
        pltpu.sync_copy(st_v, aout_s.at[pl.ds(s * rpt, rpt)])
        plsc.subcore_barrier()

        pltpu.sync_copy(src_h.at[pl.ds(g0, gmax)], si_v)
        pltpu.sync_copy(dst_h.at[pl.ds(g0, gmax)], di_v)

        def grp(j, _):
            row0 = (g0 + j) * gp
            d1 = pltpu.async_copy(xs_h.at[si_v.at[j]], gs_v, sem)
            d2 = pltpu.async_copy(xd_h.at[di_v.at[j]], gd_v, sem)
            pltpu.sync_copy(eb_h.at[pl.ds(row0, gp)], eb_v)
            d1.wait()
            d2.wait()

            def rowf(r, _):
                for t in range(8):
                    i = r * 8 + t
                    v = eb_v[r, pl.ds(t * de, de)] + gs_v[i, :] + gd_v[i, :]
                    v = jnp.maximum(v, 0.0)
                    eb_v[r, pl.ds(t * de, de)] = v
                    en_v[i, :] = v
                return 0
            lax.fori_loop(0, gp, rowf, 0)

            pltpu.sync_copy(eb_v, enew_h.at[pl.ds(row0, gp)])
            pltpu.sync_copy(en_v, ain_s.at[di_v.at[j]], add=True)
            pltpu.sync_copy(en_v, aout_s.at[si_v.at[j]], add=True)
            return 0
        lax.fori_loop(0, ng, grp, 0)
        plsc.subcore_barrier()

        pltpu.sync_copy(ain_s.at[pl.ds(s * rpt, rpt)], st_v)
        pltpu.sync_copy(st_v, ain_h.at[c, pl.ds(s * rpt, rpt)])
        pltpu.sync_copy(aout_s.at[pl.ds(s * rpt, rpt)], st_v)
        pltpu.sync_copy(st_v, aout_h.at[c, pl.ds(s * rpt, rpt)])

    return body(xs, xd, e_base, src2, dst2)


def _node_body(n_nodes, n_edges,
               x_ref, ai_ref, ao_ref, wvx_ref, wvi_ref, wvo_ref, u_ref,
               wvu_ref, bv_ref, wgx_ref, wge_ref, wgu_ref, bg_ref,
               xn_ref, un_ref, accx, acce):
    i = pl.program_id(0)
    nb = pl.num_programs(0)

    @pl.when(i == 0)
    def _():
        accx[...] = jnp.zeros_like(accx)
        acce[...] = jnp.zeros_like(acce)

    a_in = ai_ref[0] + ai_ref[1]
    a_out = ao_ref[0] + ao_ref[1]
    cv = jnp.dot(u_ref[...], wvu_ref[...], preferred_element_type=jnp.float32)
    h = (jnp.dot(x_ref[...], wvx_ref[...], preferred_element_type=jnp.float32)
         + jnp.dot(a_in, wvi_ref[...], preferred_element_type=jnp.float32)
         + jnp.dot(a_out, wvo_ref[...], preferred_element_type=jnp.float32)
         + cv + bv_ref[...])
    xn = jnp.maximum(h, 0.0)
    xn_ref[...] = xn
    accx[...] += jnp.sum(xn, axis=0, keepdims=True)
    acce[...] += jnp.sum(a_in, axis=0, keepdims=True)

    @pl.when(i == nb - 1)
    def _():
        mx = accx[...] * (1.0 / n_nodes)
        me = acce[...] * (1.0 / n_edges)
        gg = (jnp.dot(mx, wgx_ref[...], preferred_element_type=jnp.float32)
              + jnp.dot(me, wge_ref[...], preferred_element_type=jnp.float32)
              + jnp.dot(u_ref[...], wgu_ref[...], preferred_element_type=jnp.float32)
              + bg_ref[...])
        un_ref[...] = jnp.maximum(gg, 0.0)


def _node_global(x, ain2, aout2, Wvx, Wvi, Wvo, u2, Wvu, bv2,
                 Wgx, Wge, Wgu, bg2, n_edges):
    n, d = x.shape
    de = Wvi.shape[0]
    du = u2.shape[1]
    nb = 5
    r = n // nb
    const = lambda i: (0, 0)
    return pl.pallas_call(
        functools.partial(_node_body, n, n_edges),
        grid=(nb,),
        in_specs=[
            pl.BlockSpec((r, d), lambda i: (i, 0)),
            pl.BlockSpec((NC, r, de), lambda i: (0, i, 0)),
            pl.BlockSpec((NC, r, de), lambda i: (0, i, 0)),
            pl.BlockSpec((d, d), const),
            pl.BlockSpec((de, d), const),
            pl.BlockSpec((de, d), const),
            pl.BlockSpec((1, du), const),
            pl.BlockSpec((du, d), const),
            pl.BlockSpec((1, d), const),
            pl.BlockSpec((d, du), const),
            pl.BlockSpec((de, du), const),
            pl.BlockSpec((du, du), const),
            pl.BlockSpec((1, du), const),
        ],
        out_specs=[
            pl.BlockSpec((r, d), lambda i: (i, 0)),
            pl.BlockSpec((1, du), const),
        ],
        out_shape=[
            jax.ShapeDtypeStruct((n, d), jnp.float32),
            jax.ShapeDtypeStruct((1, du), jnp.float32),
        ],
        scratch_shapes=[
            pltpu.VMEM((1, d), jnp.float32),
            pltpu.VMEM((1, de), jnp.float32),
        ],
    )(x, ain2, aout2, Wvx, Wvi, Wvo, u2, Wvu, bv2, Wgx, Wge, Wgu, bg2)


def kernel(x, edge_attr, edge_index, u, We, be, Wv, bv, Wg, bg):
    n, d = x.shape
    e, de = edge_attr.shape
    du = u.shape[0]

    Wee, Wes, Wed, Weu = We[:de], We[de:de + d], We[de + d:de + 2 * d], We[de + 2 * d:]
    Wvx, Wvi, Wvo, Wvu = Wv[:d], Wv[d:d + de], Wv[d + de:d + 2 * de], Wv[d + 2 * de:]
    Wgx, Wge, Wgu = Wg[:d], Wg[d:d + de], Wg[d + de:]
    u2 = u.reshape(1, du)
    be2 = be.reshape(1, de)
    bv2 = bv.reshape(1, d)
    bg2 = bg.reshape(1, du)

    g = e // GSZ
    gmax = ((g + NW - 1) // NW + 7) // 8 * 8
    pad = gmax * NW - g
    src2 = jnp.pad(edge_index[0].reshape(g, GSZ), ((0, pad), (0, 0)))
    dst2 = jnp.pad(edge_index[1].reshape(g, GSZ), ((0, pad), (0, 0)))

    ea2 = edge_attr.reshape(e * de // 128, 128)
    W8 = jnp.kron(jnp.eye(8, dtype=jnp.float32), Wee)
    Weu8 = jnp.tile(Weu, (1, 8))
    be8 = jnp.tile(be2, (1, 8))

    xs, xd = _node_proj(x, Wes, Wed)
    e_base = _edge_base(ea2, W8, u2, Weu8, be8)
    e_new, ain2, aout2 = _edge_sc(xs, xd, e_base, src2, dst2, n, e, de)
    e_new = e_new.reshape(e, de)
    ain2 = ain2[:, :n]
    aout2 = aout2[:, :n]
    x_new, u_new = _node_global(x, ain2, aout2, Wvx, Wvi, Wvo, u2, Wvu, bv2,
                                Wgx, Wge, Wgu, bg2, e)
    return x_new, e_new, u_new.reshape(du)
